# baseline (speedup 1.0000x reference)
import functools
import math

import jax
import jax.numpy as jnp
from jax.experimental import pallas as pl

_MAX_RATIO = abs(math.log(16.0 / 1000.0))


def _decode_kernel(y_ref, d_ref, a_ref, y_out, b_out):
    y_out[...] = y_ref[...]

    d = d_ref[...]
    a = a_ref[...]
    dx = d[:, :, 0]
    dy = d[:, :, 1]
    dw = jnp.clip(d[:, :, 2], -_MAX_RATIO, _MAX_RATIO)
    dh = jnp.clip(d[:, :, 3], -_MAX_RATIO, _MAX_RATIO)

    ax1 = a[:, 0][None, :]
    ay1 = a[:, 1][None, :]
    ax2 = a[:, 2][None, :]
    ay2 = a[:, 3][None, :]
    px = (ax1 + ax2) * 0.5
    py = (ay1 + ay2) * 0.5
    pw = ax2 - ax1
    ph = ay2 - ay1

    gw = pw * jnp.exp(dw)
    gh = ph * jnp.exp(dh)
    gx = px + pw * dx
    gy = py + ph * dy

    x1 = jnp.clip(gx - gw * 0.5, 0.0, 1.0)
    y1 = jnp.clip(gy - gh * 0.5, 0.0, 1.0)
    x2 = jnp.clip(gx + gw * 0.5, 0.0, 1.0)
    y2 = jnp.clip(gy + gh * 0.5, 0.0, 1.0)
    b_out[...] = jnp.stack([x1, y1, x2, y2], axis=-1)


@functools.partial(jax.jit, static_argnames=("tn",))
def _run(y_pred, bbox_pred, anchors, tn=1000):
    B, N, C = y_pred.shape
    grid = (N // tn,)
    return pl.pallas_call(
        _decode_kernel,
        grid=grid,
        in_specs=[
            pl.BlockSpec((B, tn, C), lambda i: (0, i, 0)),
            pl.BlockSpec((B, tn, 4), lambda i: (0, i, 0)),
            pl.BlockSpec((tn, 4), lambda i: (i, 0)),
        ],
        out_specs=[
            pl.BlockSpec((B, tn, C), lambda i: (0, i, 0)),
            pl.BlockSpec((B, tn, 4), lambda i: (0, i, 0)),
        ],
        out_shape=[
            jax.ShapeDtypeStruct((B, N, C), y_pred.dtype),
            jax.ShapeDtypeStruct((B, N, 4), bbox_pred.dtype),
        ],
    )(y_pred, bbox_pred, anchors)


def kernel(y_pred, bbox_pred, anchors):
    y_out, bbox = _run(y_pred, bbox_pred, anchors)
    return (y_out, bbox)


# speedup vs baseline: 2.1898x; 2.1898x over previous
import functools
import math

import jax
import jax.numpy as jnp
from jax.experimental import pallas as pl

_MAX_RATIO = abs(math.log(16.0 / 1000.0))


def _decode_kernel(d_ref, a_ref, o_ref):
    d = d_ref[...]
    a = a_ref[...]
    am2 = jnp.roll(a, -2, axis=1)
    ctr = (a + am2) * 0.5
    siz = am2 - a
    t = jnp.exp(jnp.clip(d, -_MAX_RATIO, _MAX_RATIO))
    gxy = ctr + siz * d
    gwh = jnp.roll(siz, 2, axis=1) * t
    xy1 = gxy - 0.5 * jnp.roll(gwh, -2, axis=1)
    xy2 = jnp.roll(gxy, 2, axis=1) + 0.5 * gwh
    lane = jax.lax.broadcasted_iota(jnp.int32, d.shape, 1)
    out = jnp.where((lane & 3) < 2, xy1, xy2)
    o_ref[...] = jnp.clip(out, 0.0, 1.0)


@functools.partial(jax.jit, static_argnames=("blocks",))
def _run(bbox_pred, anchors, blocks=5):
    B, N, _ = bbox_pred.shape
    W = 4 * N
    tw = W // blocks
    d2 = bbox_pred.reshape(B, W)
    a2 = anchors.reshape(1, W)
    out = pl.pallas_call(
        _decode_kernel,
        grid=(blocks,),
        in_specs=[
            pl.BlockSpec((B, tw), lambda i: (0, i)),
            pl.BlockSpec((1, tw), lambda i: (0, i)),
        ],
        out_specs=pl.BlockSpec((B, tw), lambda i: (0, i)),
        out_shape=jax.ShapeDtypeStruct((B, W), bbox_pred.dtype),
    )(d2, a2)
    return out.reshape(B, N, 4)


def kernel(y_pred, bbox_pred, anchors):
    return (y_pred, _run(bbox_pred, anchors))
